# baseline (speedup 1.0000x reference)
import functools

import jax
import jax.numpy as jnp
from jax import lax
from jax.experimental import pallas as pl
from jax.experimental.pallas import tpu as pltpu
from jax.experimental.pallas import tpu_sc as plsc

NC = 2
NS = 16
NW = NC * NS
CH = 256
CE = 512
NB = 128


def _sc_mesh():
    return plsc.VectorSubcoreMesh(
        core_axis_name="c", subcore_axis_name="s", num_cores=NC, num_subcores=NS
    )


def _gather_sorted(table, idx):
    e_pad = idx.shape[0]
    d = table.shape[1]
    cpw = e_pad // (CH * NW)

    def body(table_hbm, idx_hbm, out_hbm, idx_v, rows_v, sem):
        c = lax.axis_index("c")
        s = lax.axis_index("s")
        wid = s * NC + c

        def chunk(i, _):
            base = (i * NW + wid) * CH
            pltpu.sync_copy(idx_hbm.at[pl.ds(base, CH)], idx_v)
            pltpu.async_copy(table_hbm.at[idx_v], rows_v, sem).wait()
            pltpu.sync_copy(rows_v, out_hbm.at[pl.ds(base, CH)])
            return 0
        lax.fori_loop(0, cpw, chunk, 0)

    k = pl.kernel(
        body,
        out_type=jax.ShapeDtypeStruct((e_pad, d), jnp.float32),
        mesh=_sc_mesh(),
        scratch_types=[
            pltpu.VMEM((CH,), jnp.int32),
            pltpu.VMEM((CH, d), jnp.float32),
            pltpu.SemaphoreType.DMA,
        ],
    )
    return k(table, idx)


def _segred(msg, eas, dsts2d, clo, chi, nblocks, with_deg):
    d = msg.shape[1]

    def body(clo_ref, chi_ref, msg_ref, eas_ref, dst_ref, *rest):
        if with_deg:
            agg_ref, deg_ref, msg_v, ea_v, dst_v, sem_m, sem_e, sem_d = rest
        else:
            agg_ref, msg_v, ea_v, dst_v, sem_m, sem_e, sem_d = rest
        b = pl.program_id(0)
        lo = clo_ref[b]
        hi = chi_ref[b]
        agg_ref[...] = jnp.zeros((NB, d), jnp.float32)
        row0 = b * NB
        iota = lax.broadcasted_iota(jnp.int32, (NB, 1), 0) + row0

        def start(t, slot):
            pltpu.make_async_copy(msg_ref.at[pl.ds(t * CE, CE)],
                                  msg_v.at[slot], sem_m.at[slot]).start()
            pltpu.make_async_copy(eas_ref.at[pl.ds(t * CE, CE)],
                                  ea_v.at[slot], sem_e.at[slot]).start()
            pltpu.make_async_copy(dst_ref.at[t], dst_v.at[slot],
                                  sem_d.at[slot]).start()

        @pl.when(lo < hi)
        def _():
            start(lo, 0)

        def step(t, deg):
            slot = lax.rem(t - lo, 2)

            @pl.when(t + 1 < hi)
            def _():
                start(t + 1, 1 - slot)

            pltpu.make_async_copy(msg_ref.at[pl.ds(t * CE, CE)],
                                  msg_v.at[slot], sem_m.at[slot]).wait()
            pltpu.make_async_copy(eas_ref.at[pl.ds(t * CE, CE)],
                                  ea_v.at[slot], sem_e.at[slot]).wait()
            pltpu.make_async_copy(dst_ref.at[t], dst_v.at[slot],
                                  sem_d.at[slot]).wait()
            dchunk = dst_v[slot]
            oh = (dchunk[None, :] == iota).astype(jnp.float32)
            m = msg_v[slot] + ea_v[slot]
            agg_ref[...] += jnp.dot(oh, m, preferred_element_type=jnp.float32)
            return deg + jnp.sum(oh, axis=1)

        deg = lax.fori_loop(lo, hi, step, jnp.zeros((NB,), jnp.float32),
                            unroll=False)
        if with_deg:
            deg_ref[...] = deg[:, None] + jnp.zeros((NB, d), jnp.float32)

    out_shape = [jax.ShapeDtypeStruct((nblocks * NB, d), jnp.float32)]
    out_specs = [pl.BlockSpec((NB, d), lambda b: (b, 0))]
    if with_deg:
        out_shape.append(jax.ShapeDtypeStruct((nblocks * NB, d), jnp.float32))
        out_specs.append(pl.BlockSpec((NB, d), lambda b: (b, 0)))

    res = pl.pallas_call(
        body,
        grid=(nblocks,),
        in_specs=[
            pl.BlockSpec(memory_space=pltpu.SMEM),
            pl.BlockSpec(memory_space=pltpu.SMEM),
            pl.BlockSpec(memory_space=pltpu.HBM),
            pl.BlockSpec(memory_space=pltpu.HBM),
            pl.BlockSpec(memory_space=pltpu.HBM),
        ],
        out_specs=out_specs,
        out_shape=out_shape,
        scratch_shapes=[
            pltpu.VMEM((2, CE, d), jnp.float32),
            pltpu.VMEM((2, CE, d), jnp.float32),
            pltpu.VMEM((2, CE), jnp.int32),
            pltpu.SemaphoreType.DMA((2,)),
            pltpu.SemaphoreType.DMA((2,)),
            pltpu.SemaphoreType.DMA((2,)),
        ],
    )(clo, chi, msg, eas, dsts2d)
    return res if with_deg else (res[0], None)


def _node_transform(x, w, blk):
    n, kdim = x.shape
    m = w.shape[1]

    def body(x_ref, w_ref, o_ref):
        o_ref[...] = jnp.dot(x_ref[...], w_ref[...],
                             preferred_element_type=jnp.float32)

    return pl.pallas_call(
        body,
        grid=(n // blk,),
        in_specs=[
            pl.BlockSpec((blk, kdim), lambda i: (i, 0)),
            pl.BlockSpec((kdim, m), lambda i: (0, 0)),
        ],
        out_specs=pl.BlockSpec((blk, m), lambda i: (i, 0)),
        out_shape=jax.ShapeDtypeStruct((n, m), jnp.float32),
    )(x, w)


def _combine(agg, deg, hs, b, wnext, bnext, relu, blk):
    n, d = hs.shape
    m = wnext.shape[1]

    def body(agg_ref, deg_ref, hs_ref, b_ref, wn_ref, bn_ref, o_ref):
        degc = jnp.maximum(deg_ref[...][:, 0:1], 1.0)
        h = agg_ref[...] / degc + hs_ref[...] + b_ref[...]
        if relu:
            h = jnp.maximum(h, 0.0)
        o_ref[...] = jnp.dot(h, wn_ref[...],
                             preferred_element_type=jnp.float32) + bn_ref[...]

    return pl.pallas_call(
        body,
        grid=(n // blk,),
        in_specs=[
            pl.BlockSpec((blk, d), lambda i: (i, 0)),
            pl.BlockSpec((blk, d), lambda i: (i, 0)),
            pl.BlockSpec((blk, d), lambda i: (i, 0)),
            pl.BlockSpec((1, d), lambda i: (0, 0)),
            pl.BlockSpec((d, m), lambda i: (0, 0)),
            pl.BlockSpec((1, m), lambda i: (0, 0)),
        ],
        out_specs=pl.BlockSpec((blk, m), lambda i: (i, 0)),
        out_shape=jax.ShapeDtypeStruct((n, m), jnp.float32),
    )(agg, deg, hs, b, wnext, bnext)


def kernel(x, edge_index, edge_attr, Wm0, Ws0, We0, b0, Wm1, Ws1, We1, b1,
           Went, bent, Wrel, brel):
    n, d = x.shape
    e = edge_index.shape[1]
    n_ent = Went.shape[1]
    n_rel = Wrel.shape[1]

    src = edge_index[0].astype(jnp.int32)
    dst = edge_index[1].astype(jnp.int32)
    perm = jnp.argsort(dst).astype(jnp.int32)
    dsts = dst[perm]
    srcp = src[perm]

    e_pad = -(-e // (CH * NW)) * (CH * NW)
    pad = e_pad - e
    srcp = jnp.concatenate([srcp, jnp.zeros((pad,), jnp.int32)])
    permp = jnp.concatenate([perm, jnp.zeros((pad,), jnp.int32)])
    dsts = jnp.concatenate([dsts, jnp.full((pad,), n, jnp.int32)])

    nblocks = -(-n // NB)
    bnd = jnp.searchsorted(dsts, jnp.arange(nblocks + 1, dtype=jnp.int32) * NB
                           ).astype(jnp.int32)
    nch = e_pad // CE
    clo = bnd[:-1] // CE
    chi = jnp.minimum(-(-bnd[1:] // CE), nch)
    dsts2d = dsts.reshape(nch, CE)

    blk = 1000 if n % 1000 == 0 else 500

    ea_blk = 2000 if e % 2000 == 0 else 1000
    eaw0 = _node_transform(edge_attr, We0, ea_blk)
    eaw1 = _node_transform(edge_attr, We1, ea_blk)
    eaw0s = _gather_sorted(eaw0, permp)
    eaw1s = _gather_sorted(eaw1, permp)

    xw = _node_transform(x, jnp.concatenate([Wm0, Ws0], axis=1), blk)
    hm0 = xw[:, :d]
    hs0 = xw[:, d:]
    msg0 = _gather_sorted(hm0, srcp)
    agg0, deg = _segred(msg0, eaw0s, dsts2d, clo, chi, nblocks, with_deg=True)
    h1w = _combine(agg0[:n], deg[:n], hs0, b0.reshape(1, d),
                   jnp.concatenate([Wm1, Ws1], axis=1),
                   jnp.zeros((1, 2 * d), jnp.float32), relu=True, blk=blk)
    hm1 = h1w[:, :d]
    hs1 = h1w[:, d:]

    msg1 = _gather_sorted(hm1, srcp)
    agg1, _ = _segred(msg1, eaw1s, dsts2d, clo, chi, nblocks, with_deg=False)
    wout = jnp.concatenate([Went, Wrel], axis=1)
    bout = jnp.concatenate([bent, brel]).reshape(1, n_ent + n_rel)
    logits = _combine(agg1[:n], deg[:n], hs1, b1.reshape(1, d),
                      wout, bout, relu=False, blk=blk)
    return (logits[:, :n_ent], logits[:, n_ent:])


# speedup vs baseline: 1.0172x; 1.0172x over previous
import functools

import jax
import jax.numpy as jnp
from jax import lax
from jax.experimental import pallas as pl
from jax.experimental.pallas import tpu as pltpu
from jax.experimental.pallas import tpu_sc as plsc

NC = 2
NS = 16
NW = NC * NS
CH = 256
CE = 512
NB = 128


def _sc_mesh():
    return plsc.VectorSubcoreMesh(
        core_axis_name="c", subcore_axis_name="s", num_cores=NC, num_subcores=NS
    )


def _gather_sorted(table, idx):
    e_pad = idx.shape[0]
    d = table.shape[1]
    cpw = e_pad // (CH * NW)

    assert cpw % 2 == 0

    def body(table_hbm, idx_hbm, out_hbm,
             idx_a, rows_a, sia, sga, swa, idx_b, rows_b, sib, sgb, swb):
        c = lax.axis_index("c")
        s = lax.axis_index("s")
        wid = s * NC + c

        def base_of(t):
            return (t * NW + wid) * CH

        def fetch(t, idx_v, si):
            pltpu.async_copy(idx_hbm.at[pl.ds(base_of(t), CH)], idx_v, si)

        def fire(t, idx_v, rows_v, si, sg):
            pltpu.make_async_copy(idx_hbm.at[pl.ds(base_of(t), CH)],
                                  idx_v, si).wait()
            pltpu.async_copy(table_hbm.at[idx_v], rows_v, sg)

        def half(t, idx_s, rows_s, si_s, sg_s, sw_s,
                 idx_o, rows_o, si_o, sg_o, sw_o):
            @pl.when(t + 1 < cpw)
            def _():
                @pl.when(t >= 1)
                def _():
                    pltpu.make_async_copy(
                        rows_o, out_hbm.at[pl.ds(base_of(t - 1), CH)],
                        sw_o).wait()
                fire(t + 1, idx_o, rows_o, si_o, sg_o)

            pltpu.make_async_copy(table_hbm.at[idx_s], rows_s, sg_s).wait()

            @pl.when(t + 2 < cpw)
            def _():
                fetch(t + 2, idx_s, si_s)

            pltpu.async_copy(rows_s, out_hbm.at[pl.ds(base_of(t), CH)], sw_s)

        fetch(0, idx_a, sia)
        fetch(1, idx_b, sib)
        fire(0, idx_a, rows_a, sia, sga)

        def pair(tt, _):
            t0 = tt * 2
            half(t0, idx_a, rows_a, sia, sga, swa,
                 idx_b, rows_b, sib, sgb, swb)
            half(t0 + 1, idx_b, rows_b, sib, sgb, swb,
                 idx_a, rows_a, sia, sga, swa)
            return 0
        lax.fori_loop(0, cpw // 2, pair, 0)

        pltpu.make_async_copy(
            rows_a, out_hbm.at[pl.ds(base_of(cpw - 2), CH)], swa).wait()
        pltpu.make_async_copy(
            rows_b, out_hbm.at[pl.ds(base_of(cpw - 1), CH)], swb).wait()

    k = pl.kernel(
        body,
        out_type=jax.ShapeDtypeStruct((e_pad, d), jnp.float32),
        mesh=_sc_mesh(),
        scratch_types=[
            pltpu.VMEM((CH,), jnp.int32),
            pltpu.VMEM((CH, d), jnp.float32),
            pltpu.SemaphoreType.DMA,
            pltpu.SemaphoreType.DMA,
            pltpu.SemaphoreType.DMA,
            pltpu.VMEM((CH,), jnp.int32),
            pltpu.VMEM((CH, d), jnp.float32),
            pltpu.SemaphoreType.DMA,
            pltpu.SemaphoreType.DMA,
            pltpu.SemaphoreType.DMA,
        ],
    )
    return k(table, idx)


def _segred(msg, eas, dsts2d, clo, chi, nblocks, with_deg):
    d = msg.shape[1]

    def body(clo_ref, chi_ref, msg_ref, eas_ref, dst_ref, *rest):
        if with_deg:
            agg_ref, deg_ref, msg_v, ea_v, dst_v, sem_m, sem_e, sem_d = rest
        else:
            agg_ref, msg_v, ea_v, dst_v, sem_m, sem_e, sem_d = rest
        b = pl.program_id(0)
        lo = clo_ref[b]
        hi = chi_ref[b]
        agg_ref[...] = jnp.zeros((NB, d), jnp.float32)
        row0 = b * NB
        iota = lax.broadcasted_iota(jnp.int32, (NB, 1), 0) + row0

        def start(t, slot):
            pltpu.make_async_copy(msg_ref.at[pl.ds(t * CE, CE)],
                                  msg_v.at[slot], sem_m.at[slot]).start()
            pltpu.make_async_copy(eas_ref.at[pl.ds(t * CE, CE)],
                                  ea_v.at[slot], sem_e.at[slot]).start()
            pltpu.make_async_copy(dst_ref.at[t], dst_v.at[slot],
                                  sem_d.at[slot]).start()

        @pl.when(lo < hi)
        def _():
            start(lo, 0)

        def step(t, deg):
            slot = lax.rem(t - lo, 2)

            @pl.when(t + 1 < hi)
            def _():
                start(t + 1, 1 - slot)

            pltpu.make_async_copy(msg_ref.at[pl.ds(t * CE, CE)],
                                  msg_v.at[slot], sem_m.at[slot]).wait()
            pltpu.make_async_copy(eas_ref.at[pl.ds(t * CE, CE)],
                                  ea_v.at[slot], sem_e.at[slot]).wait()
            pltpu.make_async_copy(dst_ref.at[t], dst_v.at[slot],
                                  sem_d.at[slot]).wait()
            dchunk = dst_v[slot]
            oh = (dchunk[None, :] == iota).astype(jnp.float32)
            m = msg_v[slot] + ea_v[slot]
            agg_ref[...] += jnp.dot(oh, m, preferred_element_type=jnp.float32)
            return deg + jnp.sum(oh, axis=1)

        deg = lax.fori_loop(lo, hi, step, jnp.zeros((NB,), jnp.float32),
                            unroll=False)
        if with_deg:
            deg_ref[...] = deg[:, None] + jnp.zeros((NB, d), jnp.float32)

    out_shape = [jax.ShapeDtypeStruct((nblocks * NB, d), jnp.float32)]
    out_specs = [pl.BlockSpec((NB, d), lambda b: (b, 0))]
    if with_deg:
        out_shape.append(jax.ShapeDtypeStruct((nblocks * NB, d), jnp.float32))
        out_specs.append(pl.BlockSpec((NB, d), lambda b: (b, 0)))

    res = pl.pallas_call(
        body,
        grid=(nblocks,),
        in_specs=[
            pl.BlockSpec(memory_space=pltpu.SMEM),
            pl.BlockSpec(memory_space=pltpu.SMEM),
            pl.BlockSpec(memory_space=pltpu.HBM),
            pl.BlockSpec(memory_space=pltpu.HBM),
            pl.BlockSpec(memory_space=pltpu.HBM),
        ],
        out_specs=out_specs,
        out_shape=out_shape,
        scratch_shapes=[
            pltpu.VMEM((2, CE, d), jnp.float32),
            pltpu.VMEM((2, CE, d), jnp.float32),
            pltpu.VMEM((2, CE), jnp.int32),
            pltpu.SemaphoreType.DMA((2,)),
            pltpu.SemaphoreType.DMA((2,)),
            pltpu.SemaphoreType.DMA((2,)),
        ],
    )(clo, chi, msg, eas, dsts2d)
    return res if with_deg else (res[0], None)


def _node_transform(x, w, blk):
    n, kdim = x.shape
    m = w.shape[1]

    def body(x_ref, w_ref, o_ref):
        o_ref[...] = jnp.dot(x_ref[...], w_ref[...],
                             preferred_element_type=jnp.float32)

    return pl.pallas_call(
        body,
        grid=(n // blk,),
        in_specs=[
            pl.BlockSpec((blk, kdim), lambda i: (i, 0)),
            pl.BlockSpec((kdim, m), lambda i: (0, 0)),
        ],
        out_specs=pl.BlockSpec((blk, m), lambda i: (i, 0)),
        out_shape=jax.ShapeDtypeStruct((n, m), jnp.float32),
    )(x, w)


def _combine(agg, deg, hs, b, wnext, bnext, relu, blk):
    n, d = hs.shape
    m = wnext.shape[1]

    def body(agg_ref, deg_ref, hs_ref, b_ref, wn_ref, bn_ref, o_ref):
        degc = jnp.maximum(deg_ref[...][:, 0:1], 1.0)
        h = agg_ref[...] / degc + hs_ref[...] + b_ref[...]
        if relu:
            h = jnp.maximum(h, 0.0)
        o_ref[...] = jnp.dot(h, wn_ref[...],
                             preferred_element_type=jnp.float32) + bn_ref[...]

    return pl.pallas_call(
        body,
        grid=(n // blk,),
        in_specs=[
            pl.BlockSpec((blk, d), lambda i: (i, 0)),
            pl.BlockSpec((blk, d), lambda i: (i, 0)),
            pl.BlockSpec((blk, d), lambda i: (i, 0)),
            pl.BlockSpec((1, d), lambda i: (0, 0)),
            pl.BlockSpec((d, m), lambda i: (0, 0)),
            pl.BlockSpec((1, m), lambda i: (0, 0)),
        ],
        out_specs=pl.BlockSpec((blk, m), lambda i: (i, 0)),
        out_shape=jax.ShapeDtypeStruct((n, m), jnp.float32),
    )(agg, deg, hs, b, wnext, bnext)


def kernel(x, edge_index, edge_attr, Wm0, Ws0, We0, b0, Wm1, Ws1, We1, b1,
           Went, bent, Wrel, brel):
    n, d = x.shape
    e = edge_index.shape[1]
    n_ent = Went.shape[1]
    n_rel = Wrel.shape[1]

    src = edge_index[0].astype(jnp.int32)
    dst = edge_index[1].astype(jnp.int32)
    perm = jnp.argsort(dst).astype(jnp.int32)
    dsts = dst[perm]
    srcp = src[perm]

    e_pad = -(-e // (CH * NW)) * (CH * NW)
    pad = e_pad - e
    srcp = jnp.concatenate([srcp, jnp.zeros((pad,), jnp.int32)])
    permp = jnp.concatenate([perm, jnp.zeros((pad,), jnp.int32)])
    dsts = jnp.concatenate([dsts, jnp.full((pad,), n, jnp.int32)])

    nblocks = -(-n // NB)
    bnd = jnp.searchsorted(dsts, jnp.arange(nblocks + 1, dtype=jnp.int32) * NB
                           ).astype(jnp.int32)
    nch = e_pad // CE
    clo = bnd[:-1] // CE
    chi = jnp.minimum(-(-bnd[1:] // CE), nch)
    dsts2d = dsts.reshape(nch, CE)

    blk = 1000 if n % 1000 == 0 else 500

    ea_blk = 2000 if e % 2000 == 0 else 1000
    eaw0 = _node_transform(edge_attr, We0, ea_blk)
    eaw1 = _node_transform(edge_attr, We1, ea_blk)
    eaw0s = _gather_sorted(eaw0, permp)
    eaw1s = _gather_sorted(eaw1, permp)

    xw = _node_transform(x, jnp.concatenate([Wm0, Ws0], axis=1), blk)
    hm0 = xw[:, :d]
    hs0 = xw[:, d:]
    msg0 = _gather_sorted(hm0, srcp)
    agg0, deg = _segred(msg0, eaw0s, dsts2d, clo, chi, nblocks, with_deg=True)
    h1w = _combine(agg0[:n], deg[:n], hs0, b0.reshape(1, d),
                   jnp.concatenate([Wm1, Ws1], axis=1),
                   jnp.zeros((1, 2 * d), jnp.float32), relu=True, blk=blk)
    hm1 = h1w[:, :d]
    hs1 = h1w[:, d:]

    msg1 = _gather_sorted(hm1, srcp)
    agg1, _ = _segred(msg1, eaw1s, dsts2d, clo, chi, nblocks, with_deg=False)
    wout = jnp.concatenate([Went, Wrel], axis=1)
    bout = jnp.concatenate([bent, brel]).reshape(1, n_ent + n_rel)
    logits = _combine(agg1[:n], deg[:n], hs1, b1.reshape(1, d),
                      wout, bout, relu=False, blk=blk)
    return (logits[:, :n_ent], logits[:, n_ent:])
